# baseline (speedup 1.0000x reference)
import functools

import jax
import jax.numpy as jnp
from jax import lax
from jax.experimental import pallas as pl
from jax.experimental.pallas import tpu as pltpu
from jax.experimental.pallas import tpu_sc as plsc

B, N, T, H, W = 16, 3000, 10, 200, 200
NUM = N // 3
LANES = 16
HALF = NUM // 2
PH = 512
NCH = PH // LANES
PT = PH * T
HW = H * W
BIG = 1 << 30

_mesh = plsc.VectorSubcoreMesh(core_axis_name="c", subcore_axis_name="s")


@functools.partial(
    pl.kernel,
    out_type=jax.ShapeDtypeStruct((B * 2 * LANES,), jnp.float32),
    mesh=_mesh,
    compiler_params=pltpu.CompilerParams(needs_layout_passes=False),
    scratch_types=[
        pltpu.VMEM((HALF * 3 * T,), jnp.float32),
        pltpu.VMEM((PT,), jnp.int32),
        pltpu.VMEM((PT,), jnp.int32),
        pltpu.VMEM((PT,), jnp.int32),
        pltpu.VMEM((PT,), jnp.float32),
        pltpu.VMEM((PT,), jnp.float32),
        pltpu.VMEM((PT,), jnp.float32),
        pltpu.VMEM((PT,), jnp.float32),
        pltpu.VMEM((PH,), jnp.float32),
        pltpu.VMEM((PH,), jnp.float32),
        pltpu.VMEM((PH,), jnp.float32),
        pltpu.VMEM((PH,), jnp.float32),
        pltpu.VMEM((B,), jnp.int32),
        pltpu.VMEM((2 * B,), jnp.float32),
        pltpu.VMEM((LANES,), jnp.float32),
        pltpu.SemaphoreType.DMA,
        pltpu.SemaphoreType.DMA,
        pltpu.SemaphoreType.DMA,
        pltpu.SemaphoreType.DMA,
    ],
)
def _plan_phase1(trajs_hbm, cost_hbm, sem_hbm, hd_hbm, cmd_hbm, tp_hbm,
                 rec_hbm,
                 traj_ref, idxt_ref, idxh_ref, idxd_ref,
                 vc_ref, vs_ref, vl_ref, vd_ref,
                 xe_ref, ye_ref, cs_ref, ss_ref,
                 cmd_ref, tp_ref, rec_ref,
                 sem_c, sem_s, sem_l, sem_d):
    cidx = lax.axis_index("c")
    sidx = lax.axis_index("s")
    b = cidx * (B // 2) + sidx // 2
    half = sidx % 2
    nbase = half * HALF
    lane = lax.iota(jnp.int32, LANES)

    pltpu.sync_copy(cmd_hbm, cmd_ref)
    pltpu.sync_copy(tp_hbm, tp_ref)
    bvec = jnp.full((LANES,), b, jnp.int32)
    cmd = plsc.load_gather(cmd_ref, [bvec])[0]
    toff = pl.multiple_of((b * N + cmd * NUM + nbase) * 3 * T, 8)
    pltpu.sync_copy(trajs_hbm.at[pl.ds(toff, HALF * 3 * T)], traj_ref)

    bofft = b * (T * HW)
    boffh = b * (2 * HW)

    def t_body(t, carry):
        tvec = jnp.full((LANES,), 3 * t, jnp.int32)

        def c_body(ci, carry2):
            r = jnp.minimum(ci * LANES + lane, HALF - 1)
            x = plsc.load_gather(traj_ref, [r * (3 * T) + tvec])
            y = plsc.load_gather(traj_ref, [r * (3 * T) + tvec + 1])
            xi = jnp.clip((x * W).astype(jnp.int32), 0, W - 1)
            yi = jnp.clip((y * H).astype(jnp.int32), 0, H - 1)
            flat = yi * W + xi
            pos = t * PH + ci * LANES
            idxt_ref[pl.ds(pos, LANES)] = bofft + t * HW + flat
            idxh_ref[pl.ds(pos, LANES)] = boffh + flat
            idxd_ref[pl.ds(pos, LANES)] = boffh + HW + flat

            @pl.when(t == T - 1)
            def _():
                xe_ref[pl.ds(ci * LANES, LANES)] = x
                ye_ref[pl.ds(ci * LANES, LANES)] = y

            return carry2

        return lax.fori_loop(0, NCH, c_body, carry)

    lax.fori_loop(0, T, t_body, 0)

    cp_c = pltpu.async_copy(cost_hbm.at[idxt_ref], vc_ref, sem_c)
    cp_s = pltpu.async_copy(sem_hbm.at[idxt_ref], vs_ref, sem_s)
    cp_l = pltpu.async_copy(hd_hbm.at[idxh_ref], vl_ref, sem_l)
    cp_d = pltpu.async_copy(hd_hbm.at[idxd_ref], vd_ref, sem_d)
    cp_c.wait()
    cp_s.wait()
    cp_l.wait()
    cp_d.wait()

    tpx = plsc.load_gather(tp_ref, [2 * bvec])[0]
    tpy = plsc.load_gather(tp_ref, [2 * bvec + 1])[0]

    def acc_body(ci, carry):
        pos0 = ci * LANES

        def t_acc(t, ac):
            a, ss = ac
            p = t * PH + pos0
            cv = vc_ref[pl.ds(p, LANES)]
            sv = vs_ref[pl.ds(p, LANES)]
            lv = vl_ref[pl.ds(p, LANES)]
            dv = vd_ref[pl.ds(p, LANES)]
            return (a + (cv + 5.0 * sv + 2.0 * lv - 3.0 * dv), ss + sv)

        zero = jnp.zeros((LANES,), jnp.float32)
        a, ss = lax.fori_loop(0, T, t_acc, (zero, zero))
        dx = xe_ref[pl.ds(pos0, LANES)] - tpx
        dy = ye_ref[pl.ds(pos0, LANES)] - tpy
        cs_ref[pl.ds(pos0, LANES)] = a + dx * dx + dy * dy
        ss_ref[pl.ds(pos0, LANES)] = ss
        return carry

    lax.fori_loop(0, NCH, acc_body, 0)

    def min_body(ci, m):
        v = cs_ref[pl.ds(ci * LANES, LANES)]
        return jnp.minimum(m, jnp.min(v))

    m = lax.fori_loop(0, NCH, min_body, jnp.float32(jnp.inf))

    def sel_body(ci, cur):
        v = cs_ref[pl.ds(ci * LANES, LANES)]
        gid = nbase + jnp.minimum(ci * LANES + lane, HALF - 1)
        cand = jnp.where(v == m, gid, jnp.int32(BIG))
        return jnp.minimum(cur, jnp.min(cand))

    sel = lax.fori_loop(0, NCH, sel_body, jnp.int32(BIG))

    def saf_body(ci, acc):
        sv = ss_ref[pl.ds(ci * LANES, LANES)]
        gid = nbase + ci * LANES + lane
        return acc + jnp.sum(jnp.where(gid == sel, sv, 0.0))

    safety = lax.fori_loop(0, NCH, saf_body, jnp.float32(0.0))

    rec = jnp.where(lane == 0, m, jnp.float32(0.0))
    rec = jnp.where(lane == 1, safety, rec)
    rec = jnp.where(lane == 2,
                    plsc.bitcast(jnp.full((LANES,), sel, jnp.int32),
                                 jnp.float32), rec)
    rec_ref[...] = rec
    pltpu.sync_copy(rec_ref, rec_hbm.at[pl.ds((b * 2 + half) * LANES, LANES)])


@functools.partial(
    pl.kernel,
    out_type=jax.ShapeDtypeStruct((B, 32), jnp.float32),
    mesh=_mesh,
    compiler_params=pltpu.CompilerParams(needs_layout_passes=False),
    scratch_types=[
        pltpu.VMEM((2 * LANES,), jnp.float32),
        pltpu.VMEM((B,), jnp.int32),
        pltpu.VMEM((40,), jnp.float32),
        pltpu.VMEM((32,), jnp.float32),
    ],
)
def _plan_phase2(trajs_hbm, cmd_hbm, rec_hbm, out_hbm,
                 rr_ref, cmd_ref, rfetch_ref, row_ref):
    cidx = lax.axis_index("c")
    sidx = lax.axis_index("s")

    @pl.when(cidx == 0)
    def _body():
        b = sidx
        lane = lax.iota(jnp.int32, LANES)

        pltpu.sync_copy(cmd_hbm, cmd_ref)
        pltpu.sync_copy(rec_hbm.at[pl.ds(b * 2 * LANES, 2 * LANES)], rr_ref)
        bvec = jnp.full((LANES,), b, jnp.int32)
        cmd = plsc.load_gather(cmd_ref, [bvec])[0]

        r0 = rr_ref[pl.ds(0, LANES)]
        r1 = rr_ref[pl.ds(LANES, LANES)]
        m0 = r0[0]
        m1 = r1[0]
        better = m1 < m0
        m2 = jnp.where(better, m1, m0)
        saf2 = jnp.where(better, r1[1], r0[1])
        sel2 = jnp.where(better,
                         plsc.bitcast(r1, jnp.int32)[2],
                         plsc.bitcast(r0, jnp.int32)[2])

        elem = (b * N + cmd * NUM + sel2) * (3 * T)
        start8 = pl.multiple_of((elem >> 3) << 3, 8)
        sh = elem - start8
        pltpu.sync_copy(trajs_hbm.at[pl.ds(start8, 40)], rfetch_ref)
        shv = jnp.full((LANES,), sh, jnp.int32)
        lo = plsc.load_gather(rfetch_ref, [shv + lane])
        hi = plsc.load_gather(
            rfetch_ref, [shv + jnp.minimum(lane + LANES, 3 * T - 1)])
        hi = jnp.where(lane == 3 * T - LANES, m2, hi)
        hi = jnp.where(lane == 3 * T + 1 - LANES, saf2, hi)
        row_ref[pl.ds(0, LANES)] = lo
        row_ref[pl.ds(LANES, LANES)] = hi
        pltpu.sync_copy(row_ref, out_hbm.at[b])


def kernel(cam_front, trajs, gt_trajs, cost_volume, semantic_pred, hd_map,
           commands, target_points, k):
    kf = jnp.asarray(k, jnp.float32).reshape(())
    trajs_flat = trajs.reshape(-1) * kf
    cost_flat = cost_volume.reshape(-1) * kf
    sem_flat = semantic_pred.reshape(-1) * kf
    hd_flat = hd_map.reshape(-1) * kf
    cmds = commands.astype(jnp.int32)
    tp_flat = target_points.reshape(-1)
    recs = _plan_phase1(trajs_flat, cost_flat, sem_flat, hd_flat,
                        cmds, tp_flat)
    return _plan_phase2(trajs_flat, cmds, recs)
